# Initial kernel scaffold; baseline (speedup 1.0000x reference)
import jax
import jax.numpy as jnp
from jax.experimental import pallas as pl


def kernel(x, edge_index, W, a):
    raise NotImplementedError("write your pallas kernel here")


# speedup vs baseline: 8.0427x; 8.0427x over previous
import functools

import jax
import jax.numpy as jnp
from jax import lax
from jax.experimental import pallas as pl
from jax.experimental.pallas import tpu as pltpu
from jax.experimental.pallas import tpu_sc as plsc

N = 10000
E = 320000
F = 128
ALPHA = 0.2

NC = 2
NS = 16
NW = NC * NS
CH = 128
NCHUNK = 79
EPT = CH * NCHUNK
EPAD = NW * EPT
NPAD = 10112
STRIPE = NPAD // NS


def _dense_body(x_ref, wt_ref, a1_ref, a2_ref, h_ref, s1_ref, s2_ref,
                m1_ref, m2_ref):
    h = jnp.dot(x_ref[...], wt_ref[...], preferred_element_type=jnp.float32)
    h_ref[...] = h
    s1 = jnp.dot(h, a1_ref[...], preferred_element_type=jnp.float32)
    s2 = jnp.dot(h, a2_ref[...], preferred_element_type=jnp.float32)
    s1_ref[...] = s1
    s2_ref[...] = s2

    @pl.when(pl.program_id(0) == 0)
    def _():
        m1_ref[0, 0] = -jnp.inf
        m2_ref[0, 0] = -jnp.inf

    m1_ref[0, 0] = jnp.maximum(m1_ref[0, 0], jnp.max(s1))
    m2_ref[0, 0] = jnp.maximum(m2_ref[0, 0], jnp.max(s2))


_RB = 1000

_dense = pl.pallas_call(
    _dense_body,
    grid=(N // _RB,),
    in_specs=[
        pl.BlockSpec((_RB, F), lambda i: (i, 0)),
        pl.BlockSpec((F, F), lambda i: (0, 0)),
        pl.BlockSpec((F, 1), lambda i: (0, 0)),
        pl.BlockSpec((F, 1), lambda i: (0, 0)),
    ],
    out_specs=[
        pl.BlockSpec((_RB, F), lambda i: (i, 0)),
        pl.BlockSpec((_RB, 1), lambda i: (i, 0)),
        pl.BlockSpec((_RB, 1), lambda i: (i, 0)),
        pl.BlockSpec((1, 1), lambda i: (0, 0), memory_space=pltpu.SMEM),
        pl.BlockSpec((1, 1), lambda i: (0, 0), memory_space=pltpu.SMEM),
    ],
    out_shape=[
        jax.ShapeDtypeStruct((N, F), jnp.float32),
        jax.ShapeDtypeStruct((N, 1), jnp.float32),
        jax.ShapeDtypeStruct((N, 1), jnp.float32),
        jax.ShapeDtypeStruct((1, 1), jnp.float32),
        jax.ShapeDtypeStruct((1, 1), jnp.float32),
    ],
)


def _edges_body(src_hbm, dst_hbm, s1_hbm, s2_hbm, m_hbm, h_hbm, z_hbm,
                num_hbm, den_hbm,
                src_v, dst_v, s1_v, s2_v, m_v, hbuf, wbuf, den_v, acc,
                sem_g, sem_s):
    cid = lax.axis_index("c")
    sid = lax.axis_index("s")
    wid = sid * NC + cid
    lane = lax.iota(jnp.int32, 16)
    zi16 = jnp.zeros((16,), jnp.int32)
    zf16 = jnp.zeros((16,), jnp.float32)

    pltpu.sync_copy(s1_hbm, s1_v)
    pltpu.sync_copy(s2_hbm, s2_v)
    pltpu.sync_copy(m_hbm, m_v)
    mvec = m_v[...]

    @pl.loop(0, NPAD // 16)
    def _zden(j):
        plsc.store_scatter(den_v, [j * 16 + lane], zf16)

    pltpu.sync_copy(z_hbm, hbuf)
    zbase = sid * STRIPE
    for q in range(STRIPE // CH):
        pltpu.sync_copy(hbuf, acc.at[pl.ds(zbase + q * CH, CH)])
    _rem = STRIPE % CH
    if _rem:
        pltpu.sync_copy(hbuf.at[pl.ds(0, _rem)],
                        acc.at[pl.ds(zbase + (STRIPE // CH) * CH, _rem)])
    plsc.subcore_barrier()

    @pl.loop(0, NCHUNK)
    def _chunk(g):
        pltpu.sync_copy(src_hbm.at[wid, g], src_v.at[0])
        pltpu.sync_copy(dst_hbm.at[wid, g], dst_v.at[0])
        pltpu.async_copy(h_hbm.at[src_v.at[0]], hbuf, sem_g).wait()

        @pl.loop(0, CH // 16)
        def _wgrp(t):
            si = plsc.load_gather(src_v, [zi16, t * 16 + lane])
            di = plsc.load_gather(dst_v, [zi16, t * 16 + lane])
            l = plsc.load_gather(s1_v, [si]) + plsc.load_gather(s2_v, [di])
            l = jnp.where(l > 0.0, l, ALPHA * l)
            plsc.store_scatter(wbuf, [t * 16 + lane], jnp.exp(l - mvec))

        @pl.loop(0, CH)
        def _erow(j):
            jv = zi16 + j
            wspl = plsc.load_gather(wbuf, [jv])
            for k in range(F // 16):
                v = plsc.load_gather(hbuf, [jv, k * 16 + lane])
                plsc.store_scatter(hbuf, [jv, k * 16 + lane], v * wspl)
            dj = plsc.load_gather(dst_v, [zi16, jv])
            plsc.addupdate_scatter(den_v, [dj], wspl, mask=lane == 0)

        pltpu.async_copy(hbuf, acc.at[dst_v.at[0]], sem_s, add=True).wait()

    plsc.subcore_barrier()
    pltpu.sync_copy(acc.at[pl.ds(zbase, STRIPE)],
                    num_hbm.at[cid, pl.ds(zbase, STRIPE)])
    pltpu.sync_copy(den_v, den_hbm.at[wid])


_edges = functools.partial(
    pl.kernel,
    out_type=[
        jax.ShapeDtypeStruct((NC, NPAD, F), jnp.float32),
        jax.ShapeDtypeStruct((NW, NPAD), jnp.float32),
    ],
    mesh=plsc.VectorSubcoreMesh(core_axis_name="c", subcore_axis_name="s"),
    compiler_params=pltpu.CompilerParams(needs_layout_passes=False),
    scratch_types=[
        pltpu.VMEM((1, CH), jnp.int32),
        pltpu.VMEM((1, CH), jnp.int32),
        pltpu.VMEM((NPAD,), jnp.float32),
        pltpu.VMEM((NPAD,), jnp.float32),
        pltpu.VMEM((16,), jnp.float32),
        pltpu.VMEM((CH, F), jnp.float32),
        pltpu.VMEM((CH,), jnp.float32),
        pltpu.VMEM((NPAD,), jnp.float32),
        pltpu.VMEM_SHARED((NPAD, F), jnp.float32),
        pltpu.SemaphoreType.DMA,
        pltpu.SemaphoreType.DMA,
    ],
)(_edges_body)


def _finish_body(num_ref, den_ref, o_ref):
    num = num_ref[0] + num_ref[1]
    den = jnp.sum(den_ref[...], axis=1)
    r = num / (den[:, None] + 1e-16)
    o_ref[...] = jnp.where(r > 0.0, r, jnp.exp(jnp.minimum(r, 0.0)) - 1.0)


_CB = 1000

_finish = pl.pallas_call(
    _finish_body,
    grid=(N // _CB,),
    in_specs=[
        pl.BlockSpec((NC, _CB, F), lambda i: (0, i, 0)),
        pl.BlockSpec((_CB, NW), lambda i: (i, 0)),
    ],
    out_specs=pl.BlockSpec((_CB, F), lambda i: (i, 0)),
    out_shape=jax.ShapeDtypeStruct((N, F), jnp.float32),
)


def kernel(x, edge_index, W, a):
    h, s1, s2, m1, m2 = _dense(x, W.T, a[:F], a[F:])
    mtot = m1[0, 0] + m2[0, 0]
    bound = jnp.where(mtot > 0.0, mtot, ALPHA * mtot)
    mvec = jnp.full((16,), bound, jnp.float32)

    src = edge_index[0]
    dst = edge_index[1]
    pad = EPAD - E
    srcp = jnp.concatenate([src, jnp.zeros((pad,), jnp.int32)]).reshape(
        NW, NCHUNK, CH)
    dstp = jnp.concatenate([dst, jnp.full((pad,), N, jnp.int32)]).reshape(
        NW, NCHUNK, CH)
    spad = jnp.zeros((NPAD - N,), jnp.float32)
    s1p = jnp.concatenate([s1.reshape(N), spad])
    s2p = jnp.concatenate([s2.reshape(N), spad])
    zeros = jnp.zeros((CH, F), jnp.float32)

    num, den = _edges(srcp, dstp, s1p, s2p, mvec, h, zeros)
    return _finish(num, den.T)


# speedup vs baseline: 9.5776x; 1.1909x over previous
import functools

import jax
import jax.numpy as jnp
from jax import lax
from jax.experimental import pallas as pl
from jax.experimental.pallas import tpu as pltpu
from jax.experimental.pallas import tpu_sc as plsc

N = 10000
E = 320000
F = 128
ALPHA = 0.2

NC = 2
NS = 16
NW = NC * NS
CH = 64
NCHUNK = 158
EPT = CH * NCHUNK
EPAD = NW * EPT
NPAD = 10112
STRIPE = NPAD // NS


def _dense_body(x_ref, wt_ref, a1_ref, a2_ref, h_ref, s1_ref, s2_ref,
                m1_ref, m2_ref):
    h = jnp.dot(x_ref[...], wt_ref[...], preferred_element_type=jnp.float32)
    h_ref[...] = h
    s1 = jnp.dot(h, a1_ref[...], preferred_element_type=jnp.float32)
    s2 = jnp.dot(h, a2_ref[...], preferred_element_type=jnp.float32)
    s1_ref[...] = s1
    s2_ref[...] = s2

    @pl.when(pl.program_id(0) == 0)
    def _():
        m1_ref[0, 0] = -jnp.inf
        m2_ref[0, 0] = -jnp.inf

    m1_ref[0, 0] = jnp.maximum(m1_ref[0, 0], jnp.max(s1))
    m2_ref[0, 0] = jnp.maximum(m2_ref[0, 0], jnp.max(s2))


_RB = 1000

_dense = pl.pallas_call(
    _dense_body,
    grid=(N // _RB,),
    in_specs=[
        pl.BlockSpec((_RB, F), lambda i: (i, 0)),
        pl.BlockSpec((F, F), lambda i: (0, 0)),
        pl.BlockSpec((F, 1), lambda i: (0, 0)),
        pl.BlockSpec((F, 1), lambda i: (0, 0)),
    ],
    out_specs=[
        pl.BlockSpec((_RB, F), lambda i: (i, 0)),
        pl.BlockSpec((_RB, 1), lambda i: (i, 0)),
        pl.BlockSpec((_RB, 1), lambda i: (i, 0)),
        pl.BlockSpec((1, 1), lambda i: (0, 0), memory_space=pltpu.SMEM),
        pl.BlockSpec((1, 1), lambda i: (0, 0), memory_space=pltpu.SMEM),
    ],
    out_shape=[
        jax.ShapeDtypeStruct((N, F), jnp.float32),
        jax.ShapeDtypeStruct((N, 1), jnp.float32),
        jax.ShapeDtypeStruct((N, 1), jnp.float32),
        jax.ShapeDtypeStruct((1, 1), jnp.float32),
        jax.ShapeDtypeStruct((1, 1), jnp.float32),
    ],
)


def _edges_body(src_hbm, dst_hbm, s1_hbm, s2_hbm, m_hbm, h_hbm, z_hbm,
                num_hbm, den_hbm,
                src0, dst0, src1, dst1, s1_v, s2_v, m_v, hbuf0, hbuf1,
                wbuf, den_v, acc, sem_g0, sem_g1, sem_s0, sem_s1):
    cid = lax.axis_index("c")
    sid = lax.axis_index("s")
    wid = sid * NC + cid
    lane = lax.iota(jnp.int32, 16)
    zi16 = jnp.zeros((16,), jnp.int32)
    zf16 = jnp.zeros((16,), jnp.float32)

    pltpu.sync_copy(s1_hbm, s1_v)
    pltpu.sync_copy(s2_hbm, s2_v)
    pltpu.sync_copy(m_hbm, m_v)
    mvec = m_v[...]

    @pl.loop(0, NPAD // 16)
    def _zden(j):
        plsc.store_scatter(den_v, [j * 16 + lane], zf16)

    pltpu.sync_copy(z_hbm, hbuf0)
    zbase = sid * STRIPE
    for q in range(STRIPE // CH):
        pltpu.sync_copy(hbuf0, acc.at[pl.ds(zbase + q * CH, CH)])
    _rem = STRIPE % CH
    if _rem:
        pltpu.sync_copy(hbuf0.at[pl.ds(0, _rem)],
                        acc.at[pl.ds(zbase + (STRIPE // CH) * CH, _rem)])
    plsc.subcore_barrier()

    def _compute(src_v, dst_v, hb):
        @pl.loop(0, CH // 16)
        def _wgrp(t):
            si = plsc.load_gather(src_v, [zi16, t * 16 + lane])
            di = plsc.load_gather(dst_v, [zi16, t * 16 + lane])
            l = plsc.load_gather(s1_v, [si]) + plsc.load_gather(s2_v, [di])
            l = jnp.where(l > 0.0, l, ALPHA * l)
            plsc.store_scatter(wbuf, [t * 16 + lane], jnp.exp(l - mvec))

        @pl.loop(0, CH)
        def _erow(j):
            jv = zi16 + j
            wspl = plsc.load_gather(wbuf, [jv])
            for k in range(F // 16):
                v = plsc.load_gather(hb, [jv, k * 16 + lane])
                plsc.store_scatter(hb, [jv, k * 16 + lane], v * wspl)
            dj = plsc.load_gather(dst_v, [zi16, jv])
            plsc.addupdate_scatter(den_v, [dj], wspl, mask=lane == 0)

    pltpu.sync_copy(src_hbm.at[wid, 0], src0.at[0])
    pltpu.sync_copy(dst_hbm.at[wid, 0], dst0.at[0])
    pltpu.async_copy(h_hbm.at[src0.at[0]], hbuf0, sem_g0)

    @pl.loop(0, NCHUNK // 2)
    def _pair(t):
        ge = 2 * t
        pltpu.make_async_copy(h_hbm.at[src0.at[0]], hbuf0, sem_g0).wait()

        @pl.when(t > 0)
        def _():
            pltpu.make_async_copy(hbuf1, acc.at[dst1.at[0]], sem_s1).wait()
        pltpu.sync_copy(src_hbm.at[wid, ge + 1], src1.at[0])
        pltpu.sync_copy(dst_hbm.at[wid, ge + 1], dst1.at[0])
        pltpu.async_copy(h_hbm.at[src1.at[0]], hbuf1, sem_g1)
        _compute(src0, dst0, hbuf0)
        pltpu.async_copy(hbuf0, acc.at[dst0.at[0]], sem_s0, add=True)

        pltpu.make_async_copy(h_hbm.at[src1.at[0]], hbuf1, sem_g1).wait()
        pltpu.make_async_copy(hbuf0, acc.at[dst0.at[0]], sem_s0).wait()

        @pl.when(t < NCHUNK // 2 - 1)
        def _():
            pltpu.sync_copy(src_hbm.at[wid, ge + 2], src0.at[0])
            pltpu.sync_copy(dst_hbm.at[wid, ge + 2], dst0.at[0])
            pltpu.async_copy(h_hbm.at[src0.at[0]], hbuf0, sem_g0)
        _compute(src1, dst1, hbuf1)
        pltpu.async_copy(hbuf1, acc.at[dst1.at[0]], sem_s1, add=True)

    pltpu.make_async_copy(hbuf1, acc.at[dst1.at[0]], sem_s1).wait()

    plsc.subcore_barrier()
    pltpu.sync_copy(acc.at[pl.ds(zbase, STRIPE)],
                    num_hbm.at[cid, pl.ds(zbase, STRIPE)])
    pltpu.sync_copy(den_v, den_hbm.at[wid])


_edges = functools.partial(
    pl.kernel,
    out_type=[
        jax.ShapeDtypeStruct((NC, NPAD, F), jnp.float32),
        jax.ShapeDtypeStruct((NW, NPAD), jnp.float32),
    ],
    mesh=plsc.VectorSubcoreMesh(core_axis_name="c", subcore_axis_name="s"),
    compiler_params=pltpu.CompilerParams(needs_layout_passes=False),
    scratch_types=[
        pltpu.VMEM((1, CH), jnp.int32),
        pltpu.VMEM((1, CH), jnp.int32),
        pltpu.VMEM((1, CH), jnp.int32),
        pltpu.VMEM((1, CH), jnp.int32),
        pltpu.VMEM((NPAD,), jnp.float32),
        pltpu.VMEM((NPAD,), jnp.float32),
        pltpu.VMEM((16,), jnp.float32),
        pltpu.VMEM((CH, F), jnp.float32),
        pltpu.VMEM((CH, F), jnp.float32),
        pltpu.VMEM((CH,), jnp.float32),
        pltpu.VMEM((NPAD,), jnp.float32),
        pltpu.VMEM_SHARED((NPAD, F), jnp.float32),
        pltpu.SemaphoreType.DMA,
        pltpu.SemaphoreType.DMA,
        pltpu.SemaphoreType.DMA,
        pltpu.SemaphoreType.DMA,
    ],
)(_edges_body)


def _finish_body(num_ref, den_ref, o_ref):
    num = num_ref[0] + num_ref[1]
    den = jnp.sum(den_ref[...], axis=1)
    r = num / (den[:, None] + 1e-16)
    o_ref[...] = jnp.where(r > 0.0, r, jnp.exp(jnp.minimum(r, 0.0)) - 1.0)


_CB = 1000

_finish = pl.pallas_call(
    _finish_body,
    grid=(N // _CB,),
    in_specs=[
        pl.BlockSpec((NC, _CB, F), lambda i: (0, i, 0)),
        pl.BlockSpec((_CB, NW), lambda i: (i, 0)),
    ],
    out_specs=pl.BlockSpec((_CB, F), lambda i: (i, 0)),
    out_shape=jax.ShapeDtypeStruct((N, F), jnp.float32),
)


def kernel(x, edge_index, W, a):
    h, s1, s2, m1, m2 = _dense(x, W.T, a[:F], a[F:])
    mtot = m1[0, 0] + m2[0, 0]
    bound = jnp.where(mtot > 0.0, mtot, ALPHA * mtot)
    mvec = jnp.full((16,), bound, jnp.float32)

    src = edge_index[0]
    dst = edge_index[1]
    pad = EPAD - E
    srcp = jnp.concatenate([src, jnp.zeros((pad,), jnp.int32)]).reshape(
        NW, NCHUNK, CH)
    dstp = jnp.concatenate([dst, jnp.full((pad,), N, jnp.int32)]).reshape(
        NW, NCHUNK, CH)
    spad = jnp.zeros((NPAD - N,), jnp.float32)
    s1p = jnp.concatenate([s1.reshape(N), spad])
    s2p = jnp.concatenate([s2.reshape(N), spad])
    zeros = jnp.zeros((CH, F), jnp.float32)

    num, den = _edges(srcp, dstp, s1p, s2p, mvec, h, zeros)
    return _finish(num, den.T)


# speedup vs baseline: 14.3322x; 1.4964x over previous
import functools

import jax
import jax.numpy as jnp
from jax import lax
from jax.experimental import pallas as pl
from jax.experimental.pallas import tpu as pltpu
from jax.experimental.pallas import tpu_sc as plsc

N = 10000
E = 320000
F = 128
ALPHA = 0.2

NC = 2
NS = 16
NW = NC * NS
CH = 64
NCHUNK = 158
EPT = CH * NCHUNK
EPAD = NW * EPT
NPAD = 10112
STRIPE = NPAD // NS


def _dense_body(x_ref, wt_ref, a1_ref, a2_ref, h_ref, s1_ref, s2_ref,
                m1_ref, m2_ref):
    h = jnp.dot(x_ref[...], wt_ref[...], preferred_element_type=jnp.float32)
    h_ref[...] = h
    s1 = jnp.dot(h, a1_ref[...], preferred_element_type=jnp.float32)
    s2 = jnp.dot(h, a2_ref[...], preferred_element_type=jnp.float32)
    s1_ref[...] = s1
    s2_ref[...] = s2

    @pl.when(pl.program_id(0) == 0)
    def _():
        m1_ref[0, 0] = -jnp.inf
        m2_ref[0, 0] = -jnp.inf

    m1_ref[0, 0] = jnp.maximum(m1_ref[0, 0], jnp.max(s1))
    m2_ref[0, 0] = jnp.maximum(m2_ref[0, 0], jnp.max(s2))


_RB = 1000

_dense = pl.pallas_call(
    _dense_body,
    grid=(N // _RB,),
    in_specs=[
        pl.BlockSpec((_RB, F), lambda i: (i, 0)),
        pl.BlockSpec((F, F), lambda i: (0, 0)),
        pl.BlockSpec((F, 1), lambda i: (0, 0)),
        pl.BlockSpec((F, 1), lambda i: (0, 0)),
    ],
    out_specs=[
        pl.BlockSpec((_RB, F), lambda i: (i, 0)),
        pl.BlockSpec((_RB, 1), lambda i: (i, 0)),
        pl.BlockSpec((_RB, 1), lambda i: (i, 0)),
        pl.BlockSpec((1, 1), lambda i: (0, 0), memory_space=pltpu.SMEM),
        pl.BlockSpec((1, 1), lambda i: (0, 0), memory_space=pltpu.SMEM),
    ],
    out_shape=[
        jax.ShapeDtypeStruct((N, F), jnp.float32),
        jax.ShapeDtypeStruct((N, 1), jnp.float32),
        jax.ShapeDtypeStruct((N, 1), jnp.float32),
        jax.ShapeDtypeStruct((1, 1), jnp.float32),
        jax.ShapeDtypeStruct((1, 1), jnp.float32),
    ],
)


def _edges_body(src_hbm, dst_hbm, s1_hbm, s2_hbm, m_hbm, h_hbm, z_hbm,
                num_hbm, den_hbm,
                src0, dst0, src1, dst1, s1_v, s2_v, m_v, hbuf0, hbuf1,
                wbuf, den_v, acc, sem_g0, sem_g1, sem_s0, sem_s1):
    cid = lax.axis_index("c")
    sid = lax.axis_index("s")
    wid = sid * NC + cid
    lane = lax.iota(jnp.int32, 16)
    zi16 = jnp.zeros((16,), jnp.int32)
    zf16 = jnp.zeros((16,), jnp.float32)

    pltpu.sync_copy(s1_hbm, s1_v)
    pltpu.sync_copy(s2_hbm, s2_v)
    pltpu.sync_copy(m_hbm, m_v)
    mvec = m_v[...]

    @pl.loop(0, NPAD // 16)
    def _zden(j):
        plsc.store_scatter(den_v, [j * 16 + lane], zf16)

    pltpu.sync_copy(z_hbm, hbuf0)
    zbase = sid * STRIPE
    for q in range(STRIPE // CH):
        pltpu.sync_copy(hbuf0, acc.at[pl.ds(zbase + q * CH, CH)])
    _rem = STRIPE % CH
    if _rem:
        pltpu.sync_copy(hbuf0.at[pl.ds(0, _rem)],
                        acc.at[pl.ds(zbase + (STRIPE // CH) * CH, _rem)])
    plsc.subcore_barrier()

    def _compute(src_v, dst_v, hb):
        @plsc.parallel_loop(0, CH // 16, unroll=2)
        def _wgrp(t):
            si = plsc.load_gather(src_v, [zi16, t * 16 + lane])
            di = plsc.load_gather(dst_v, [zi16, t * 16 + lane])
            l = plsc.load_gather(s1_v, [si]) + plsc.load_gather(s2_v, [di])
            l = jnp.where(l > 0.0, l, ALPHA * l)
            plsc.store_scatter(wbuf, [t * 16 + lane], jnp.exp(l - mvec))

        @plsc.parallel_loop(0, CH, unroll=4)
        def _erow(j):
            jv = zi16 + j
            wspl = plsc.load_gather(wbuf, [jv])
            for k in range(F // 16):
                v = plsc.load_gather(hb, [jv, k * 16 + lane])
                plsc.store_scatter(hb, [jv, k * 16 + lane], v * wspl)
            dj = plsc.load_gather(dst_v, [zi16, jv])
            plsc.addupdate_scatter(den_v, [dj], wspl, mask=lane == 0)

    pltpu.sync_copy(src_hbm.at[wid, 0], src0.at[0])
    pltpu.sync_copy(dst_hbm.at[wid, 0], dst0.at[0])
    pltpu.async_copy(h_hbm.at[src0.at[0]], hbuf0, sem_g0)

    @pl.loop(0, NCHUNK // 2)
    def _pair(t):
        ge = 2 * t
        pltpu.make_async_copy(h_hbm.at[src0.at[0]], hbuf0, sem_g0).wait()

        @pl.when(t > 0)
        def _():
            pltpu.make_async_copy(hbuf1, acc.at[dst1.at[0]], sem_s1).wait()
        pltpu.sync_copy(src_hbm.at[wid, ge + 1], src1.at[0])
        pltpu.sync_copy(dst_hbm.at[wid, ge + 1], dst1.at[0])
        pltpu.async_copy(h_hbm.at[src1.at[0]], hbuf1, sem_g1)
        _compute(src0, dst0, hbuf0)
        pltpu.async_copy(hbuf0, acc.at[dst0.at[0]], sem_s0, add=True)

        pltpu.make_async_copy(h_hbm.at[src1.at[0]], hbuf1, sem_g1).wait()
        pltpu.make_async_copy(hbuf0, acc.at[dst0.at[0]], sem_s0).wait()

        @pl.when(t < NCHUNK // 2 - 1)
        def _():
            pltpu.sync_copy(src_hbm.at[wid, ge + 2], src0.at[0])
            pltpu.sync_copy(dst_hbm.at[wid, ge + 2], dst0.at[0])
            pltpu.async_copy(h_hbm.at[src0.at[0]], hbuf0, sem_g0)
        _compute(src1, dst1, hbuf1)
        pltpu.async_copy(hbuf1, acc.at[dst1.at[0]], sem_s1, add=True)

    pltpu.make_async_copy(hbuf1, acc.at[dst1.at[0]], sem_s1).wait()

    plsc.subcore_barrier()
    pltpu.sync_copy(acc.at[pl.ds(zbase, STRIPE)],
                    num_hbm.at[cid, pl.ds(zbase, STRIPE)])
    pltpu.sync_copy(den_v, den_hbm.at[wid])


_edges = functools.partial(
    pl.kernel,
    out_type=[
        jax.ShapeDtypeStruct((NC, NPAD, F), jnp.float32),
        jax.ShapeDtypeStruct((NW, NPAD), jnp.float32),
    ],
    mesh=plsc.VectorSubcoreMesh(core_axis_name="c", subcore_axis_name="s"),
    compiler_params=pltpu.CompilerParams(needs_layout_passes=False),
    scratch_types=[
        pltpu.VMEM((1, CH), jnp.int32),
        pltpu.VMEM((1, CH), jnp.int32),
        pltpu.VMEM((1, CH), jnp.int32),
        pltpu.VMEM((1, CH), jnp.int32),
        pltpu.VMEM((NPAD,), jnp.float32),
        pltpu.VMEM((NPAD,), jnp.float32),
        pltpu.VMEM((16,), jnp.float32),
        pltpu.VMEM((CH, F), jnp.float32),
        pltpu.VMEM((CH, F), jnp.float32),
        pltpu.VMEM((CH,), jnp.float32),
        pltpu.VMEM((NPAD,), jnp.float32),
        pltpu.VMEM_SHARED((NPAD, F), jnp.float32),
        pltpu.SemaphoreType.DMA,
        pltpu.SemaphoreType.DMA,
        pltpu.SemaphoreType.DMA,
        pltpu.SemaphoreType.DMA,
    ],
)(_edges_body)


def _finish_body(num_ref, den_ref, o_ref):
    num = num_ref[0] + num_ref[1]
    den = jnp.sum(den_ref[...], axis=1)
    r = num / (den[:, None] + 1e-16)
    o_ref[...] = jnp.where(r > 0.0, r, jnp.exp(jnp.minimum(r, 0.0)) - 1.0)


_CB = 1000

_finish = pl.pallas_call(
    _finish_body,
    grid=(N // _CB,),
    in_specs=[
        pl.BlockSpec((NC, _CB, F), lambda i: (0, i, 0)),
        pl.BlockSpec((_CB, NW), lambda i: (i, 0)),
    ],
    out_specs=pl.BlockSpec((_CB, F), lambda i: (i, 0)),
    out_shape=jax.ShapeDtypeStruct((N, F), jnp.float32),
)


def kernel(x, edge_index, W, a):
    h, s1, s2, m1, m2 = _dense(x, W.T, a[:F], a[F:])
    mtot = m1[0, 0] + m2[0, 0]
    bound = jnp.where(mtot > 0.0, mtot, ALPHA * mtot)
    mvec = jnp.full((16,), bound, jnp.float32)

    src = edge_index[0]
    dst = edge_index[1]
    pad = EPAD - E
    srcp = jnp.concatenate([src, jnp.zeros((pad,), jnp.int32)]).reshape(
        NW, NCHUNK, CH)
    dstp = jnp.concatenate([dst, jnp.full((pad,), N, jnp.int32)]).reshape(
        NW, NCHUNK, CH)
    spad = jnp.zeros((NPAD - N,), jnp.float32)
    s1p = jnp.concatenate([s1.reshape(N), spad])
    s2p = jnp.concatenate([s2.reshape(N), spad])
    zeros = jnp.zeros((CH, F), jnp.float32)

    num, den = _edges(srcp, dstp, s1p, s2p, mvec, h, zeros)
    return _finish(num, den.T)


# speedup vs baseline: 14.3585x; 1.0018x over previous
import functools

import jax
import jax.numpy as jnp
from jax import lax
from jax.experimental import pallas as pl
from jax.experimental.pallas import tpu as pltpu
from jax.experimental.pallas import tpu_sc as plsc

N = 10000
E = 320000
F = 128
ALPHA = 0.2

NC = 2
NS = 16
NW = NC * NS
CH = 64
NCHUNK = 158
EPT = CH * NCHUNK
EPAD = NW * EPT
NPAD = 10112
STRIPE = NPAD // NS


def _dense_body(x_ref, wt_ref, a1_ref, a2_ref, h_ref, s1_ref, s2_ref,
                m1_ref, m2_ref):
    h = jnp.dot(x_ref[...], wt_ref[...], preferred_element_type=jnp.float32)
    h_ref[...] = h
    s1 = jnp.dot(h, a1_ref[...], preferred_element_type=jnp.float32)
    s2 = jnp.dot(h, a2_ref[...], preferred_element_type=jnp.float32)
    s1_ref[...] = s1
    s2_ref[...] = s2

    @pl.when(pl.program_id(0) == 0)
    def _():
        m1_ref[0, 0] = -jnp.inf
        m2_ref[0, 0] = -jnp.inf

    m1_ref[0, 0] = jnp.maximum(m1_ref[0, 0], jnp.max(s1))
    m2_ref[0, 0] = jnp.maximum(m2_ref[0, 0], jnp.max(s2))


_RB = 1000

_dense = pl.pallas_call(
    _dense_body,
    grid=(N // _RB,),
    in_specs=[
        pl.BlockSpec((_RB, F), lambda i: (i, 0)),
        pl.BlockSpec((F, F), lambda i: (0, 0)),
        pl.BlockSpec((F, 1), lambda i: (0, 0)),
        pl.BlockSpec((F, 1), lambda i: (0, 0)),
    ],
    out_specs=[
        pl.BlockSpec((_RB, F), lambda i: (i, 0)),
        pl.BlockSpec((_RB, 1), lambda i: (i, 0)),
        pl.BlockSpec((_RB, 1), lambda i: (i, 0)),
        pl.BlockSpec((1, 1), lambda i: (0, 0), memory_space=pltpu.SMEM),
        pl.BlockSpec((1, 1), lambda i: (0, 0), memory_space=pltpu.SMEM),
    ],
    out_shape=[
        jax.ShapeDtypeStruct((N, F), jnp.float32),
        jax.ShapeDtypeStruct((N, 1), jnp.float32),
        jax.ShapeDtypeStruct((N, 1), jnp.float32),
        jax.ShapeDtypeStruct((1, 1), jnp.float32),
        jax.ShapeDtypeStruct((1, 1), jnp.float32),
    ],
)


def _edges_body(src_hbm, dst_hbm, s1_hbm, s2_hbm, m_hbm, h_hbm, z_hbm,
                num_hbm, den_hbm,
                src0, dst0, src1, dst1, s1_v, s2_v, m_v, hbuf0, hbuf1,
                wbuf, den_v, acc, sem_g0, sem_g1, sem_s0, sem_s1):
    cid = lax.axis_index("c")
    sid = lax.axis_index("s")
    wid = sid * NC + cid
    lane = lax.iota(jnp.int32, 16)
    zi16 = jnp.zeros((16,), jnp.int32)
    zf16 = jnp.zeros((16,), jnp.float32)

    pltpu.sync_copy(s1_hbm, s1_v)
    pltpu.sync_copy(s2_hbm, s2_v)
    pltpu.sync_copy(m_hbm, m_v)
    mvec = m_v[...]

    @pl.loop(0, NPAD // 16)
    def _zden(j):
        plsc.store_scatter(den_v, [j * 16 + lane], zf16)

    pltpu.sync_copy(z_hbm, hbuf0)
    zbase = sid * STRIPE
    for q in range(STRIPE // CH):
        pltpu.sync_copy(hbuf0, acc.at[pl.ds(zbase + q * CH, CH)])
    _rem = STRIPE % CH
    if _rem:
        pltpu.sync_copy(hbuf0.at[pl.ds(0, _rem)],
                        acc.at[pl.ds(zbase + (STRIPE // CH) * CH, _rem)])
    plsc.subcore_barrier()

    def _compute(src_v, dst_v, hb):
        @plsc.parallel_loop(0, CH // 16, unroll=2)
        def _wgrp(t):
            si = plsc.load_gather(src_v, [zi16, t * 16 + lane])
            di = plsc.load_gather(dst_v, [zi16, t * 16 + lane])
            l = plsc.load_gather(s1_v, [si]) + plsc.load_gather(s2_v, [di])
            l = jnp.where(l > 0.0, l, ALPHA * l)
            plsc.store_scatter(wbuf, [t * 16 + lane], jnp.exp(l - mvec))

        @plsc.parallel_loop(0, CH, unroll=8)
        def _erow(j):
            jv = zi16 + j
            wspl = plsc.load_gather(wbuf, [jv])
            for k in range(F // 16):
                v = plsc.load_gather(hb, [jv, k * 16 + lane])
                plsc.store_scatter(hb, [jv, k * 16 + lane], v * wspl)
            dj = plsc.load_gather(dst_v, [zi16, jv])
            plsc.addupdate_scatter(den_v, [dj], wspl, mask=lane == 0)

    pltpu.sync_copy(src_hbm.at[wid, 0], src0.at[0])
    pltpu.sync_copy(dst_hbm.at[wid, 0], dst0.at[0])
    pltpu.async_copy(h_hbm.at[src0.at[0]], hbuf0, sem_g0)

    @pl.loop(0, NCHUNK // 2)
    def _pair(t):
        ge = 2 * t
        pltpu.make_async_copy(h_hbm.at[src0.at[0]], hbuf0, sem_g0).wait()

        @pl.when(t > 0)
        def _():
            pltpu.make_async_copy(hbuf1, acc.at[dst1.at[0]], sem_s1).wait()
        pltpu.sync_copy(src_hbm.at[wid, ge + 1], src1.at[0])
        pltpu.sync_copy(dst_hbm.at[wid, ge + 1], dst1.at[0])
        pltpu.async_copy(h_hbm.at[src1.at[0]], hbuf1, sem_g1)
        _compute(src0, dst0, hbuf0)
        pltpu.async_copy(hbuf0, acc.at[dst0.at[0]], sem_s0, add=True)

        pltpu.make_async_copy(h_hbm.at[src1.at[0]], hbuf1, sem_g1).wait()
        pltpu.make_async_copy(hbuf0, acc.at[dst0.at[0]], sem_s0).wait()

        @pl.when(t < NCHUNK // 2 - 1)
        def _():
            pltpu.sync_copy(src_hbm.at[wid, ge + 2], src0.at[0])
            pltpu.sync_copy(dst_hbm.at[wid, ge + 2], dst0.at[0])
            pltpu.async_copy(h_hbm.at[src0.at[0]], hbuf0, sem_g0)
        _compute(src1, dst1, hbuf1)
        pltpu.async_copy(hbuf1, acc.at[dst1.at[0]], sem_s1, add=True)

    pltpu.make_async_copy(hbuf1, acc.at[dst1.at[0]], sem_s1).wait()

    plsc.subcore_barrier()
    pltpu.sync_copy(acc.at[pl.ds(zbase, STRIPE)],
                    num_hbm.at[cid, pl.ds(zbase, STRIPE)])
    pltpu.sync_copy(den_v, den_hbm.at[wid])


_edges = functools.partial(
    pl.kernel,
    out_type=[
        jax.ShapeDtypeStruct((NC, NPAD, F), jnp.float32),
        jax.ShapeDtypeStruct((NW, NPAD), jnp.float32),
    ],
    mesh=plsc.VectorSubcoreMesh(core_axis_name="c", subcore_axis_name="s"),
    compiler_params=pltpu.CompilerParams(needs_layout_passes=False),
    scratch_types=[
        pltpu.VMEM((1, CH), jnp.int32),
        pltpu.VMEM((1, CH), jnp.int32),
        pltpu.VMEM((1, CH), jnp.int32),
        pltpu.VMEM((1, CH), jnp.int32),
        pltpu.VMEM((NPAD,), jnp.float32),
        pltpu.VMEM((NPAD,), jnp.float32),
        pltpu.VMEM((16,), jnp.float32),
        pltpu.VMEM((CH, F), jnp.float32),
        pltpu.VMEM((CH, F), jnp.float32),
        pltpu.VMEM((CH,), jnp.float32),
        pltpu.VMEM((NPAD,), jnp.float32),
        pltpu.VMEM_SHARED((NPAD, F), jnp.float32),
        pltpu.SemaphoreType.DMA,
        pltpu.SemaphoreType.DMA,
        pltpu.SemaphoreType.DMA,
        pltpu.SemaphoreType.DMA,
    ],
)(_edges_body)


def _finish_body(num_ref, den_ref, o_ref):
    num = num_ref[0] + num_ref[1]
    den = jnp.sum(den_ref[...], axis=1)
    r = num / (den[:, None] + 1e-16)
    o_ref[...] = jnp.where(r > 0.0, r, jnp.exp(jnp.minimum(r, 0.0)) - 1.0)


_CB = 1000

_finish = pl.pallas_call(
    _finish_body,
    grid=(N // _CB,),
    in_specs=[
        pl.BlockSpec((NC, _CB, F), lambda i: (0, i, 0)),
        pl.BlockSpec((_CB, NW), lambda i: (i, 0)),
    ],
    out_specs=pl.BlockSpec((_CB, F), lambda i: (i, 0)),
    out_shape=jax.ShapeDtypeStruct((N, F), jnp.float32),
)


def kernel(x, edge_index, W, a):
    h, s1, s2, m1, m2 = _dense(x, W.T, a[:F], a[F:])
    mtot = m1[0, 0] + m2[0, 0]
    bound = jnp.where(mtot > 0.0, mtot, ALPHA * mtot)
    mvec = jnp.full((16,), bound, jnp.float32)

    src = edge_index[0]
    dst = edge_index[1]
    pad = EPAD - E
    srcp = jnp.concatenate([src, jnp.zeros((pad,), jnp.int32)]).reshape(
        NW, NCHUNK, CH)
    dstp = jnp.concatenate([dst, jnp.full((pad,), N, jnp.int32)]).reshape(
        NW, NCHUNK, CH)
    spad = jnp.zeros((NPAD - N,), jnp.float32)
    s1p = jnp.concatenate([s1.reshape(N), spad])
    s2p = jnp.concatenate([s2.reshape(N), spad])
    zeros = jnp.zeros((CH, F), jnp.float32)

    num, den = _edges(srcp, dstp, s1p, s2p, mvec, h, zeros)
    return _finish(num, den.T)


# speedup vs baseline: 17.7574x; 1.2367x over previous
import functools

import jax
import jax.numpy as jnp
from jax import lax
from jax.experimental import pallas as pl
from jax.experimental.pallas import tpu as pltpu
from jax.experimental.pallas import tpu_sc as plsc

N = 10000
E = 320000
F = 128
ALPHA = 0.2

NC = 2
NS = 16
NW = NC * NS
CH = 48
NCHUNK = 210
EPT = CH * NCHUNK
EPAD = NW * EPT
NPAD = 10112
STRIPE = NPAD // NS


def _dense_body(x_ref, wt_ref, a1_ref, a2_ref, h_ref, s1_ref, s2_ref,
                m1_ref, m2_ref):
    h = jnp.dot(x_ref[...], wt_ref[...], preferred_element_type=jnp.float32)
    h_ref[...] = h
    s1 = jnp.dot(h, a1_ref[...], preferred_element_type=jnp.float32)
    s2 = jnp.dot(h, a2_ref[...], preferred_element_type=jnp.float32)
    s1_ref[...] = s1
    s2_ref[...] = s2

    @pl.when(pl.program_id(0) == 0)
    def _():
        m1_ref[0, 0] = -jnp.inf
        m2_ref[0, 0] = -jnp.inf

    m1_ref[0, 0] = jnp.maximum(m1_ref[0, 0], jnp.max(s1))
    m2_ref[0, 0] = jnp.maximum(m2_ref[0, 0], jnp.max(s2))


_RB = 1000

_dense = pl.pallas_call(
    _dense_body,
    grid=(N // _RB,),
    in_specs=[
        pl.BlockSpec((_RB, F), lambda i: (i, 0)),
        pl.BlockSpec((F, F), lambda i: (0, 0)),
        pl.BlockSpec((F, 1), lambda i: (0, 0)),
        pl.BlockSpec((F, 1), lambda i: (0, 0)),
    ],
    out_specs=[
        pl.BlockSpec((_RB, F), lambda i: (i, 0)),
        pl.BlockSpec((_RB, 1), lambda i: (i, 0)),
        pl.BlockSpec((_RB, 1), lambda i: (i, 0)),
        pl.BlockSpec((1, 1), lambda i: (0, 0), memory_space=pltpu.SMEM),
        pl.BlockSpec((1, 1), lambda i: (0, 0), memory_space=pltpu.SMEM),
    ],
    out_shape=[
        jax.ShapeDtypeStruct((N, F), jnp.float32),
        jax.ShapeDtypeStruct((N, 1), jnp.float32),
        jax.ShapeDtypeStruct((N, 1), jnp.float32),
        jax.ShapeDtypeStruct((1, 1), jnp.float32),
        jax.ShapeDtypeStruct((1, 1), jnp.float32),
    ],
)


def _edges_body(src_hbm, dst_hbm, s1_hbm, s2_hbm, m_hbm, h_hbm, z_hbm,
                num_hbm, den_hbm,
                src0, dst0, src1, dst1, src2, dst2, s1_v, s2_v, m_v,
                hbuf0, hbuf1, hbuf2, wbuf, den_v, acc,
                sem_g0, sem_g1, sem_g2, sem_s0, sem_s1, sem_s2):
    cid = lax.axis_index("c")
    sid = lax.axis_index("s")
    wid = sid * NC + cid
    lane = lax.iota(jnp.int32, 16)
    zi16 = jnp.zeros((16,), jnp.int32)
    zf16 = jnp.zeros((16,), jnp.float32)

    pltpu.sync_copy(s1_hbm, s1_v)
    pltpu.sync_copy(s2_hbm, s2_v)
    pltpu.sync_copy(m_hbm, m_v)
    mvec = m_v[...]

    @pl.loop(0, NPAD // 16)
    def _zden(j):
        plsc.store_scatter(den_v, [j * 16 + lane], zf16)

    pltpu.sync_copy(z_hbm, hbuf0)
    zbase = sid * STRIPE
    for q in range(STRIPE // CH):
        pltpu.sync_copy(hbuf0, acc.at[pl.ds(zbase + q * CH, CH)])
    _rem = STRIPE % CH
    if _rem:
        pltpu.sync_copy(hbuf0.at[pl.ds(0, _rem)],
                        acc.at[pl.ds(zbase + (STRIPE // CH) * CH, _rem)])
    plsc.subcore_barrier()

    def _compute(src_v, dst_v, hb):
        @plsc.parallel_loop(0, CH // 16, unroll=3)
        def _wgrp(t):
            si = plsc.load_gather(src_v, [zi16, t * 16 + lane])
            di = plsc.load_gather(dst_v, [zi16, t * 16 + lane])
            l = plsc.load_gather(s1_v, [si]) + plsc.load_gather(s2_v, [di])
            l = jnp.where(l > 0.0, l, ALPHA * l)
            plsc.store_scatter(wbuf, [t * 16 + lane], jnp.exp(l - mvec))

        @plsc.parallel_loop(0, CH, unroll=8)
        def _erow(j):
            jv = zi16 + j
            wspl = plsc.load_gather(wbuf, [jv])
            for k in range(F // 16):
                v = plsc.load_gather(hb, [jv, k * 16 + lane])
                plsc.store_scatter(hb, [jv, k * 16 + lane], v * wspl)
            dj = plsc.load_gather(dst_v, [zi16, jv])
            plsc.addupdate_scatter(den_v, [dj], wspl, mask=lane == 0)

    bufs = ((src0, dst0, hbuf0, sem_g0, sem_s0),
            (src1, dst1, hbuf1, sem_g1, sem_s1),
            (src2, dst2, hbuf2, sem_g2, sem_s2))

    for b in range(2):
        sv, dv, hb, sg, _ = bufs[b]
        pltpu.sync_copy(src_hbm.at[wid, b], sv.at[0])
        pltpu.sync_copy(dst_hbm.at[wid, b], dv.at[0])
        pltpu.async_copy(h_hbm.at[sv.at[0]], hb, sg)

    @pl.loop(0, NCHUNK // 3)
    def _trip(t):
        for b in range(3):
            g = 3 * t + b
            sv, dv, hb, sg, ss = bufs[b]
            sv2, dv2, hb2, sg2, ss2 = bufs[(b + 2) % 3]
            pltpu.make_async_copy(h_hbm.at[sv.at[0]], hb, sg).wait()
            _compute(sv, dv, hb)
            pltpu.async_copy(hb, acc.at[dv.at[0]], ss, add=True)

            @pl.when(g + 2 < NCHUNK)
            def _():
                @pl.when(g > 0)
                def _():
                    pltpu.make_async_copy(hb2, acc.at[dv2.at[0]],
                                          ss2).wait()
                pltpu.sync_copy(src_hbm.at[wid, g + 2], sv2.at[0])
                pltpu.sync_copy(dst_hbm.at[wid, g + 2], dv2.at[0])
                pltpu.async_copy(h_hbm.at[sv2.at[0]], hb2, sg2)

    for g in (NCHUNK - 3, NCHUNK - 2, NCHUNK - 1):
        sv, dv, hb, _, ss = bufs[g % 3]
        pltpu.make_async_copy(hb, acc.at[dv.at[0]], ss).wait()

    plsc.subcore_barrier()
    pltpu.sync_copy(acc.at[pl.ds(zbase, STRIPE)],
                    num_hbm.at[cid, pl.ds(zbase, STRIPE)])
    pltpu.sync_copy(den_v, den_hbm.at[wid])


_edges = functools.partial(
    pl.kernel,
    out_type=[
        jax.ShapeDtypeStruct((NC, NPAD, F), jnp.float32),
        jax.ShapeDtypeStruct((NW, NPAD), jnp.float32),
    ],
    mesh=plsc.VectorSubcoreMesh(core_axis_name="c", subcore_axis_name="s"),
    compiler_params=pltpu.CompilerParams(needs_layout_passes=False),
    scratch_types=[
        pltpu.VMEM((1, CH), jnp.int32),
        pltpu.VMEM((1, CH), jnp.int32),
        pltpu.VMEM((1, CH), jnp.int32),
        pltpu.VMEM((1, CH), jnp.int32),
        pltpu.VMEM((1, CH), jnp.int32),
        pltpu.VMEM((1, CH), jnp.int32),
        pltpu.VMEM((NPAD,), jnp.float32),
        pltpu.VMEM((NPAD,), jnp.float32),
        pltpu.VMEM((16,), jnp.float32),
        pltpu.VMEM((CH, F), jnp.float32),
        pltpu.VMEM((CH, F), jnp.float32),
        pltpu.VMEM((CH, F), jnp.float32),
        pltpu.VMEM((CH,), jnp.float32),
        pltpu.VMEM((NPAD,), jnp.float32),
        pltpu.VMEM_SHARED((NPAD, F), jnp.float32),
        pltpu.SemaphoreType.DMA,
        pltpu.SemaphoreType.DMA,
        pltpu.SemaphoreType.DMA,
        pltpu.SemaphoreType.DMA,
        pltpu.SemaphoreType.DMA,
        pltpu.SemaphoreType.DMA,
    ],
)(_edges_body)


def _finish_body(num_ref, den_ref, o_ref):
    num = num_ref[0] + num_ref[1]
    den = jnp.sum(den_ref[...], axis=1)
    r = num / (den[:, None] + 1e-16)
    o_ref[...] = jnp.where(r > 0.0, r, jnp.exp(jnp.minimum(r, 0.0)) - 1.0)


_CB = 1000

_finish = pl.pallas_call(
    _finish_body,
    grid=(N // _CB,),
    in_specs=[
        pl.BlockSpec((NC, _CB, F), lambda i: (0, i, 0)),
        pl.BlockSpec((_CB, NW), lambda i: (i, 0)),
    ],
    out_specs=pl.BlockSpec((_CB, F), lambda i: (i, 0)),
    out_shape=jax.ShapeDtypeStruct((N, F), jnp.float32),
)


def kernel(x, edge_index, W, a):
    h, s1, s2, m1, m2 = _dense(x, W.T, a[:F], a[F:])
    mtot = m1[0, 0] + m2[0, 0]
    bound = jnp.where(mtot > 0.0, mtot, ALPHA * mtot)
    mvec = jnp.full((16,), bound, jnp.float32)

    src = edge_index[0]
    dst = edge_index[1]
    pad = EPAD - E
    srcp = jnp.concatenate([src, jnp.zeros((pad,), jnp.int32)]).reshape(
        NW, NCHUNK, CH)
    dstp = jnp.concatenate([dst, jnp.full((pad,), N, jnp.int32)]).reshape(
        NW, NCHUNK, CH)
    spad = jnp.zeros((NPAD - N,), jnp.float32)
    s1p = jnp.concatenate([s1.reshape(N), spad])
    s2p = jnp.concatenate([s2.reshape(N), spad])
    zeros = jnp.zeros((CH, F), jnp.float32)

    num, den = _edges(srcp, dstp, s1p, s2p, mvec, h, zeros)
    return _finish(num, den.T)
